# baseline (speedup 1.0000x reference)
import functools

import jax
import jax.numpy as jnp
from jax import lax
from jax.experimental import pallas as pl
from jax.experimental.pallas import tpu as pltpu
from jax.experimental.pallas import tpu_sc as plsc

_L = 16


def kernel(neigh_values, u, v, r):
    N, NEIGH = neigh_values.shape
    K = u.shape[1]
    info = plsc.get_sparse_core_info()
    NC, NS = info.num_cores, info.num_subcores
    NW = NC * NS
    B = 1024
    assert N % (NW * B) == 0 and B % _L == 0
    sites_per_w = N // NW
    nblk = sites_per_w // B

    mesh = plsc.VectorSubcoreMesh(core_axis_name="c", subcore_axis_name="s")

    @functools.partial(
        pl.kernel,
        mesh=mesh,
        compiler_params=pltpu.CompilerParams(needs_layout_passes=False),
        out_type=(
            jax.ShapeDtypeStruct((N * K,), jnp.float32),
            jax.ShapeDtypeStruct((N,), jnp.int32),
        ),
        scratch_types=[
            pltpu.VMEM((B * NEIGH,), jnp.int32),
            pltpu.VMEM((B * K,), jnp.float32),
            pltpu.VMEM((B,), jnp.float32),
            pltpu.VMEM((B,), jnp.float32),
            pltpu.VMEM((B * K,), jnp.float32),
            pltpu.VMEM((B,), jnp.int32),
        ],
    )
    def run(neigh_hbm, u_hbm, v_hbm, r_hbm, probs_hbm, samp_hbm,
            neigh_v, u_v, v_v, r_v, probs_v, samp_v):
        wid = lax.axis_index("s") * NC + lax.axis_index("c")
        lane = jnp.arange(_L, dtype=jnp.int32)

        def blk_body(b, carry):
            base = wid * sites_per_w + b * B
            pltpu.sync_copy(neigh_hbm.at[pl.ds(base * NEIGH, B * NEIGH)], neigh_v)
            pltpu.sync_copy(u_hbm.at[pl.ds(base * K, B * K)], u_v)
            pltpu.sync_copy(v_hbm.at[pl.ds(base, B)], v_v)
            pltpu.sync_copy(r_hbm.at[pl.ds(base, B)], r_v)

            def grp_body(c, carry2):
                noff = lane * NEIGH + c * (_L * NEIGH)
                uoff = lane * K + c * (_L * K)
                packed = jnp.zeros((_L,), jnp.int32)
                for j in range(NEIGH):
                    nv = plsc.load_gather(neigh_v, [noff + j])
                    packed = packed + (jnp.ones((_L,), jnp.int32) << (nv * 4))
                v_vec = v_v[pl.ds(c * _L, _L)]
                r_vec = r_v[pl.ds(c * _L, _L)]
                pots = []
                total = None
                for k in range(K):
                    uk = plsc.load_gather(u_v, [uoff + k])
                    mk = ((packed >> (4 * k)) & 15).astype(jnp.float32)
                    pk = jnp.exp(v_vec * mk + uk)
                    pots.append(pk)
                    total = pk if total is None else total + pk
                inv = 1.0 / total
                running = jnp.zeros((_L,), jnp.float32)
                sample = jnp.zeros((_L,), jnp.int32)
                one = jnp.ones((_L,), jnp.int32)
                zero = jnp.zeros((_L,), jnp.int32)
                for k in range(K):
                    prob_k = pots[k] * inv
                    plsc.store_scatter(probs_v, [uoff + k], prob_k)
                    running = running + prob_k
                    sample = sample + jnp.where(running < r_vec, one, zero)
                samp_v[pl.ds(c * _L, _L)] = sample
                return carry2

            lax.fori_loop(0, B // _L, grp_body, 0)
            pltpu.sync_copy(probs_v, probs_hbm.at[pl.ds(base * K, B * K)])
            pltpu.sync_copy(samp_v, samp_hbm.at[pl.ds(base, B)])
            return carry

        lax.fori_loop(0, nblk, blk_body, 0)

    neigh_flat = neigh_values.astype(jnp.int32).reshape(N * NEIGH)
    probs_flat, samples = run(neigh_flat, u.reshape(N * K), v, r)
    return probs_flat.reshape(N, K), samples


# speedup vs baseline: 11.9936x; 11.9936x over previous
import functools

import jax
import jax.numpy as jnp
from jax import lax
from jax.experimental import pallas as pl
from jax.experimental.pallas import tpu as pltpu
from jax.experimental.pallas import tpu_sc as plsc

_L = 16
_G = 128


def kernel(neigh_values, u, v, r):
    N, NEIGH = neigh_values.shape
    K = u.shape[1]
    info = plsc.get_sparse_core_info()
    NC, NS = info.num_cores, info.num_subcores
    NW = NC * NS
    B = 2048
    assert N % (NW * B) == 0 and B % _G == 0
    sites_per_w = N // NW
    nblk = sites_per_w // B
    assert nblk % 2 == 0

    mesh = plsc.VectorSubcoreMesh(core_axis_name="c", subcore_axis_name="s")

    @functools.partial(
        pl.kernel,
        mesh=mesh,
        compiler_params=pltpu.CompilerParams(needs_layout_passes=False),
        out_type=(
            jax.ShapeDtypeStruct((K * N,), jnp.float32),
            jax.ShapeDtypeStruct((N,), jnp.int32),
        ),
        scratch_types=[
            pltpu.VMEM((NEIGH * B,), jnp.int32),
            pltpu.VMEM((NEIGH * B,), jnp.int32),
            pltpu.VMEM((K * B,), jnp.float32),
            pltpu.VMEM((K * B,), jnp.float32),
            pltpu.VMEM((B,), jnp.float32),
            pltpu.VMEM((B,), jnp.float32),
            pltpu.VMEM((B,), jnp.float32),
            pltpu.VMEM((B,), jnp.float32),
            pltpu.VMEM((K * B,), jnp.float32),
            pltpu.VMEM((K * B,), jnp.float32),
            pltpu.VMEM((B,), jnp.int32),
            pltpu.VMEM((B,), jnp.int32),
            pltpu.SemaphoreType.DMA,
            pltpu.SemaphoreType.DMA,
            pltpu.SemaphoreType.DMA,
            pltpu.SemaphoreType.DMA,
        ],
    )
    def run(neigh_hbm, u_hbm, v_hbm, r_hbm, probs_hbm, samp_hbm,
            neigh_v0, neigh_v1, u_v0, u_v1, v_v0, v_v1, r_v0, r_v1,
            probs_v0, probs_v1, samp_v0, samp_v1,
            sem_in0, sem_in1, sem_out0, sem_out1):
        wid = lax.axis_index("s") * NC + lax.axis_index("c")
        w0 = wid * sites_per_w
        bufs = (
            (neigh_v0, u_v0, v_v0, r_v0, probs_v0, samp_v0, sem_in0, sem_out0),
            (neigh_v1, u_v1, v_v1, r_v1, probs_v1, samp_v1, sem_in1, sem_out1),
        )
        one = jnp.ones((_L,), jnp.int32)
        zero = jnp.zeros((_L,), jnp.int32)

        def in_descs(b, buf):
            neigh_v, u_v, v_v, r_v = buf[0], buf[1], buf[2], buf[3]
            sem = buf[6]
            base = w0 + b * B
            return [
                (neigh_hbm.at[pl.ds(base * NEIGH, B * NEIGH)], neigh_v, sem),
                (u_hbm.at[pl.ds(base * K, B * K)], u_v, sem),
                (v_hbm.at[pl.ds(base, B)], v_v, sem),
                (r_hbm.at[pl.ds(base, B)], r_v, sem),
            ]

        def out_descs(b, buf):
            probs_v, samp_v = buf[4], buf[5]
            sem = buf[7]
            base = w0 + b * B
            return [
                (probs_v, probs_hbm.at[pl.ds(base * K, B * K)], sem),
                (samp_v, samp_hbm.at[pl.ds(base, B)], sem),
            ]

        def compute(buf):
            neigh_v, u_v, v_v, r_v, probs_v, samp_v = buf[:6]

            def chunk(m, carry):
                gbase = m * (_G * K)
                sbase = m * _G
                for s in range(_G // _L):
                    off = s * _L
                    packed = zero
                    for j in range(NEIGH):
                        nv = neigh_v[pl.ds(gbase + j * _G + off, _L)]
                        packed = packed + (one << (nv * 4))
                    v_vec = v_v[pl.ds(sbase + off, _L)]
                    r_vec = r_v[pl.ds(sbase + off, _L)]
                    pots = []
                    total = None
                    for k in range(K):
                        uk = u_v[pl.ds(gbase + k * _G + off, _L)]
                        mk = ((packed >> (4 * k)) & 15).astype(jnp.float32)
                        pk = jnp.exp(v_vec * mk + uk)
                        pots.append(pk)
                        total = pk if total is None else total + pk
                    inv = 1.0 / total
                    running = None
                    sample = zero
                    for k in range(K):
                        prob_k = pots[k] * inv
                        probs_v[pl.ds(gbase + k * _G + off, _L)] = prob_k
                        running = prob_k if running is None else running + prob_k
                        sample = sample + jnp.where(running < r_vec, one, zero)
                    samp_v[pl.ds(sbase + off, _L)] = sample
                return carry

            lax.fori_loop(0, B // _G, chunk, 0)

        def phase(b, cur, nxt):
            @pl.when(b + 1 < nblk)
            def _():
                for s, t, sem in in_descs(b + 1, nxt):
                    pltpu.async_copy(s, t, sem)

            for s, t, sem in in_descs(b, cur):
                pltpu.make_async_copy(s, t, sem).wait()

            @pl.when(b >= 2)
            def _():
                for s, t, sem in out_descs(b, cur):
                    pltpu.make_async_copy(s, t, sem).wait()

            compute(cur)
            for s, t, sem in out_descs(b, cur):
                pltpu.async_copy(s, t, sem)

        for s, t, sem in in_descs(0, bufs[0]):
            pltpu.async_copy(s, t, sem)

        def pair(i, carry):
            phase(2 * i, bufs[0], bufs[1])
            phase(2 * i + 1, bufs[1], bufs[0])
            return carry

        lax.fori_loop(0, nblk // 2, pair, 0)
        for bi in range(2):
            for s, t, sem in out_descs(nblk - 2 + bi, bufs[bi]):
                pltpu.make_async_copy(s, t, sem).wait()

    neigh_g = (neigh_values.astype(jnp.int32)
               .reshape(N // _G, _G, NEIGH).transpose(0, 2, 1).reshape(-1))
    u_g = u.reshape(N // _G, _G, K).transpose(0, 2, 1).reshape(-1)
    probs_flat, samples = run(neigh_g, u_g, v, r)
    probs = (probs_flat.reshape(N // _G, K, _G).transpose(0, 2, 1)
             .reshape(N, K))
    return probs, samples


# speedup vs baseline: 16.0205x; 1.3358x over previous
import functools

import jax
import jax.numpy as jnp
from jax import lax
from jax.experimental import pallas as pl
from jax.experimental.pallas import tpu as pltpu
from jax.experimental.pallas import tpu_sc as plsc

_L = 16
_G = 128
_B = 2048
_SC_BLOCKS_PER_TILE = 8
_TC_GB = 128


def _sc_part(neigh_hbm_shape_N, M, NEIGH, K):
    N = neigh_hbm_shape_N
    info = plsc.get_sparse_core_info()
    NC, NS = info.num_cores, info.num_subcores
    NW = NC * NS
    B = _B
    assert M % (NW * B) == 0
    sites_per_w = M // NW
    nblk = sites_per_w // B
    assert nblk % 2 == 0

    mesh = plsc.VectorSubcoreMesh(core_axis_name="c", subcore_axis_name="s")

    @functools.partial(
        pl.kernel,
        mesh=mesh,
        compiler_params=pltpu.CompilerParams(needs_layout_passes=False),
        out_type=(
            jax.ShapeDtypeStruct((K * N,), jnp.float32),
            jax.ShapeDtypeStruct((N,), jnp.int32),
        ),
        scratch_types=[
            pltpu.VMEM((NEIGH * B,), jnp.int32),
            pltpu.VMEM((NEIGH * B,), jnp.int32),
            pltpu.VMEM((K * B,), jnp.float32),
            pltpu.VMEM((K * B,), jnp.float32),
            pltpu.VMEM((B,), jnp.float32),
            pltpu.VMEM((B,), jnp.float32),
            pltpu.VMEM((B,), jnp.float32),
            pltpu.VMEM((B,), jnp.float32),
            pltpu.VMEM((K * B,), jnp.float32),
            pltpu.VMEM((K * B,), jnp.float32),
            pltpu.VMEM((B,), jnp.int32),
            pltpu.VMEM((B,), jnp.int32),
            pltpu.SemaphoreType.DMA,
            pltpu.SemaphoreType.DMA,
            pltpu.SemaphoreType.DMA,
            pltpu.SemaphoreType.DMA,
        ],
    )
    def run(neigh_hbm, u_hbm, v_hbm, r_hbm, probs_hbm, samp_hbm,
            neigh_v0, neigh_v1, u_v0, u_v1, v_v0, v_v1, r_v0, r_v1,
            probs_v0, probs_v1, samp_v0, samp_v1,
            sem_in0, sem_in1, sem_out0, sem_out1):
        wid = lax.axis_index("s") * NC + lax.axis_index("c")
        w0 = wid * sites_per_w
        bufs = (
            (neigh_v0, u_v0, v_v0, r_v0, probs_v0, samp_v0, sem_in0, sem_out0),
            (neigh_v1, u_v1, v_v1, r_v1, probs_v1, samp_v1, sem_in1, sem_out1),
        )
        one = jnp.ones((_L,), jnp.int32)
        zero = jnp.zeros((_L,), jnp.int32)

        def in_descs(b, buf):
            neigh_v, u_v, v_v, r_v = buf[0], buf[1], buf[2], buf[3]
            sem = buf[6]
            base = w0 + b * B
            return [
                (neigh_hbm.at[pl.ds(base * NEIGH, B * NEIGH)], neigh_v, sem),
                (u_hbm.at[pl.ds(base * K, B * K)], u_v, sem),
                (v_hbm.at[pl.ds(base, B)], v_v, sem),
                (r_hbm.at[pl.ds(base, B)], r_v, sem),
            ]

        def out_descs(b, buf):
            probs_v, samp_v = buf[4], buf[5]
            sem = buf[7]
            base = w0 + b * B
            return [
                (probs_v, probs_hbm.at[pl.ds(base * K, B * K)], sem),
                (samp_v, samp_hbm.at[pl.ds(base, B)], sem),
            ]

        def compute(buf):
            neigh_v, u_v, v_v, r_v, probs_v, samp_v = buf[:6]

            @plsc.parallel_loop(0, B // _G, 1, unroll=1)
            def chunk(m):
                gbase = m * (_G * K)
                sbase = m * _G
                for s in range(_G // _L):
                    off = s * _L
                    packed = zero
                    for j in range(NEIGH):
                        nv = neigh_v[pl.ds(gbase + j * _G + off, _L)]
                        packed = packed + (one << (nv * 4))
                    v_vec = v_v[pl.ds(sbase + off, _L)]
                    r_vec = r_v[pl.ds(sbase + off, _L)]
                    pots = []
                    total = None
                    for k in range(K):
                        uk = u_v[pl.ds(gbase + k * _G + off, _L)]
                        mk = ((packed >> (4 * k)) & 15).astype(jnp.float32)
                        pk = jnp.exp(v_vec * mk + uk)
                        pots.append(pk)
                        total = pk if total is None else total + pk
                    inv = 1.0 / total
                    running = None
                    sample = zero
                    for k in range(K):
                        prob_k = pots[k] * inv
                        probs_v[pl.ds(gbase + k * _G + off, _L)] = prob_k
                        running = prob_k if running is None else running + prob_k
                        sample = sample + jnp.where(running < r_vec, one, zero)
                    samp_v[pl.ds(sbase + off, _L)] = sample

        def phase(b, cur, nxt):
            @pl.when(b + 1 < nblk)
            def _():
                for s, t, sem in in_descs(b + 1, nxt):
                    pltpu.async_copy(s, t, sem)

            for s, t, sem in in_descs(b, cur):
                pltpu.make_async_copy(s, t, sem).wait()

            @pl.when(b >= 2)
            def _():
                for s, t, sem in out_descs(b, cur):
                    pltpu.make_async_copy(s, t, sem).wait()

            compute(cur)
            for s, t, sem in out_descs(b, cur):
                pltpu.async_copy(s, t, sem)

        for s, t, sem in in_descs(0, bufs[0]):
            pltpu.async_copy(s, t, sem)

        def pair(i, carry):
            phase(2 * i, bufs[0], bufs[1])
            phase(2 * i + 1, bufs[1], bufs[0])
            return carry

        lax.fori_loop(0, nblk // 2, pair, 0)
        for bi in range(2):
            for s, t, sem in out_descs(nblk - 2 + bi, bufs[bi]):
                pltpu.make_async_copy(s, t, sem).wait()

    return run


def _tc_part(ngroups, goff, NEIGH, K):
    GB = _TC_GB
    assert ngroups % GB == 0

    def body(neigh_ref, u_ref, v_ref, r_ref, probs_ref, samp_ref):
        nv = neigh_ref[...]
        one = jnp.ones((GB, _G), jnp.int32)
        packed = jnp.zeros((GB, _G), jnp.int32)
        for j in range(NEIGH):
            packed = packed + (one << (nv[:, j, :] * 4))
        v_b = v_ref[...]
        r_b = r_ref[...]
        pots = []
        total = None
        for k in range(K):
            mk = ((packed >> (4 * k)) & 15).astype(jnp.float32)
            pk = jnp.exp(v_b * mk + u_ref[:, k, :])
            pots.append(pk)
            total = pk if total is None else total + pk
        inv = 1.0 / total
        running = None
        sample = jnp.zeros((GB, _G), jnp.int32)
        izero = jnp.zeros((GB, _G), jnp.int32)
        for k in range(K):
            prob_k = pots[k] * inv
            probs_ref[:, k, :] = prob_k
            running = prob_k if running is None else running + prob_k
            sample = sample + jnp.where(running < r_b, one, izero)
        samp_ref[...] = sample

    gb0 = goff // GB
    return pl.pallas_call(
        body,
        grid=(ngroups // GB,),
        in_specs=[
            pl.BlockSpec((GB, NEIGH, _G), lambda g: (gb0 + g, 0, 0)),
            pl.BlockSpec((GB, K, _G), lambda g: (gb0 + g, 0, 0)),
            pl.BlockSpec((GB, _G), lambda g: (gb0 + g, 0)),
            pl.BlockSpec((GB, _G), lambda g: (gb0 + g, 0)),
        ],
        out_specs=[
            pl.BlockSpec((GB, K, _G), lambda g: (g, 0, 0)),
            pl.BlockSpec((GB, _G), lambda g: (g, 0)),
        ],
        out_shape=[
            jax.ShapeDtypeStruct((ngroups, K, _G), jnp.float32),
            jax.ShapeDtypeStruct((ngroups, _G), jnp.int32),
        ],
    )


def kernel(neigh_values, u, v, r):
    N, NEIGH = neigh_values.shape
    K = u.shape[1]
    info = plsc.get_sparse_core_info()
    NW = info.num_cores * info.num_subcores
    M = _SC_BLOCKS_PER_TILE * NW * _B
    assert 0 < M < N and (N - M) % (_TC_GB * _G) == 0 and M % _G == 0

    neigh_g = (neigh_values.astype(jnp.int32)
               .reshape(N // _G, _G, NEIGH).transpose(0, 2, 1))
    u_g = u.reshape(N // _G, _G, K).transpose(0, 2, 1)
    v_g = v.reshape(N // _G, _G)
    r_g = r.reshape(N // _G, _G)

    sc = _sc_part(N, M, NEIGH, K)
    probs_flat, samples = sc(neigh_g.reshape(-1), u_g.reshape(-1), v, r)

    tc = _tc_part(N - M > 0 and (N - M) // _G, M // _G, NEIGH, K)
    probs_tc, samp_tc = tc(neigh_g, u_g, v_g, r_g)

    probs_flat = lax.dynamic_update_slice(
        probs_flat, probs_tc.reshape(-1), (M * K,))
    samples = lax.dynamic_update_slice(samples, samp_tc.reshape(-1), (M,))

    probs = (probs_flat.reshape(N // _G, K, _G).transpose(0, 2, 1)
             .reshape(N, K))
    return probs, samples
